# baseline (speedup 1.0000x reference)
import functools

import jax
import jax.numpy as jnp
from jax import lax
from jax.experimental import pallas as pl
from jax.experimental.pallas import tpu as pltpu
from jax.experimental.pallas import tpu_sc as plsc

MARGIN = 1.0
EPS = 1e-8
CHUNK = 64


def _nrsqrt(x):
    xi = plsc.bitcast(x, jnp.int32)
    yi = jnp.int32(0x5F3759DF) - lax.shift_right_logical(xi, 1)
    y = plsc.bitcast(yi, jnp.float32)
    for _ in range(3):
        y = y * (1.5 - 0.5 * x * y * y)
    return y


@functools.lru_cache(maxsize=None)
def _make_tc_pack(E, D, bw):
    grid = (E + bw - 1) // bw
    h = bw // 2

    def body(x_ref, o_ref):
        y = jnp.swapaxes(x_ref[...], 0, 1)
        o_ref[:, 0:D] = y[0:h]
        o_ref[:, D:2 * D] = y[h:bw]

    return pl.pallas_call(
        body,
        grid=(grid,),
        in_specs=[pl.BlockSpec((D, bw), lambda i: (0, i))],
        out_specs=pl.BlockSpec((h, 2 * D), lambda i: (i, 0)),
        out_shape=jax.ShapeDtypeStruct((grid * h, 2 * D), jnp.float32),
    )


@functools.lru_cache(maxsize=None)
def _make_sc_fused(B, D):
    info = plsc.get_sparse_core_info()
    NC, NS, L = info.num_cores, info.num_subcores, info.num_lanes
    NW = NC * NS
    b_per_w = B // NW
    assert B % NW == 0 and b_per_w % CHUNK == 0 and CHUNK % L == 0
    n_ch = b_per_w // CHUNK
    n_g = CHUNK // L
    W = 2 * D
    mesh = plsc.VectorSubcoreMesh(core_axis_name="c", subcore_axis_name="s")

    row_scratch = [pltpu.VMEM((CHUNK, W), jnp.float32) for _ in range(12)]
    idx_scratch = ([pltpu.VMEM((b_per_w + L,), jnp.int32) for _ in range(6)]
                   + [pltpu.VMEM((b_per_w,), jnp.int32) for _ in range(6)])

    @functools.partial(
        pl.kernel,
        mesh=mesh,
        compiler_params=pltpu.CompilerParams(
            use_tc_tiling_on_sc=True, needs_layout_passes=False),
        out_type=jax.ShapeDtypeStruct((NW * L,), jnp.float32),
        scratch_types=idx_scratch + row_scratch + [
            pltpu.VMEM((12 * CHUNK,), jnp.float32),
            pltpu.VMEM((L,), jnp.float32),
            pltpu.SemaphoreType.DMA,
            pltpu.SemaphoreType.DMA,
        ],
    )
    def sc_fused(ph_hbm, pr_hbm, pt_hbm, nh_hbm, nr_hbm, nt_hbm,
                 ent_hbm, rel_hbm, out_hbm, *refs):
        idx_bufs = refs[0:6]
        tid_bufs = refs[6:12]
        bufs = (refs[12:18], refs[18:24])
        sums = refs[24]
        part_v = refs[25]
        sems = (refs[26], refs[27])
        wid = lax.axis_index("s") * NC + lax.axis_index("c")
        base = wid * b_per_w
        tables = (ent_hbm, rel_hbm, ent_hbm, ent_hbm, rel_hbm, ent_hbm)
        id_hbm = (ph_hbm, pr_hbm, pt_hbm, nh_hbm, nr_hbm, nt_hbm)

        for e in range(6):
            pltpu.sync_copy(id_hbm[e].at[pl.ds(base, b_per_w)],
                            idx_bufs[e].at[pl.ds(0, b_per_w)])

        def sbody(j, _):
            for e in range(6):
                v = idx_bufs[e][pl.ds(j * L, L)]
                hi = lax.shift_right_logical(v & jnp.int32(-1024), 1)
                tid_bufs[e][pl.ds(j * L, L)] = hi | (v & 511)
            return 0

        lax.fori_loop(0, b_per_w // L, sbody, 0)

        def fire(c, p):
            return [
                pltpu.async_copy(
                    tables[e].at[tid_bufs[e].at[pl.ds(c * CHUNK, CHUNK)]],
                    bufs[p][e], sems[p])
                for e in range(6)
            ]

        last = lax.iota(jnp.int32, L) == (L - 1)

        def compute(c, p, acc):
            def tbody(i, _):
                def ld(e):
                    v0 = idx_bufs[e][pl.ds(c * CHUNK + i, L)][0]
                    o = lax.shift_right_logical(v0, 3) & jnp.int32(D)
                    return [bufs[p][e][i, pl.ds(o + k * L, L)]
                            for k in range(D // L)]

                h, r, t, h2, r2, t2 = (ld(e) for e in range(6))
                he = [v + EPS for v in h]
                te = [v + EPS for v in t]
                h2e = [v + EPS for v in h2]
                t2e = [v + EPS for v in t2]

                def red(a, b):
                    s = a[0] * b[0]
                    for k in range(1, len(a)):
                        s = s + a[k] * b[k]
                    return s

                terms = (red(he, he), red(te, te), red(r, r),
                         red(he, r), red(he, te), red(r, te),
                         red(h2e, h2e), red(t2e, t2e), red(r2, r2),
                         red(h2e, r2), red(h2e, t2e), red(r2, t2e))
                iv = jnp.full((L,), 0, jnp.int32) + i
                for m, v in enumerate(terms):
                    plsc.store_scatter(sums, [iv + m * CHUNK],
                                       plsc.cumsum(v), mask=last)
                return 0

            lax.fori_loop(0, CHUNK, tbody, 0)

            def gbody(g, acc):
                o = g * L
                (sh, st, sr, chr_, cht, crt,
                 sh2, st2, sr2, chr2, cht2, crt2) = (
                     sums[pl.ds(o + m * CHUNK, L)] for m in range(12))

                def score(sa, sb, sc_, cab, cac, cbc):
                    al = _nrsqrt(jnp.maximum(sa, 1e-24))
                    be = _nrsqrt(jnp.maximum(sb, 1e-24))
                    sq = (al * al * sa + be * be * sb + sc_
                          + 2.0 * al * cab - 2.0 * al * be * cac
                          - 2.0 * be * cbc)
                    sq = jnp.maximum(sq, 0.0)
                    return sq * _nrsqrt(jnp.maximum(sq, 1e-24))

                pos = score(sh, st, sr, chr_, cht, crt)
                neg = score(sh2, st2, sr2, chr2, cht2, crt2)
                return acc + jnp.maximum(0.0, MARGIN + pos - neg)

            return lax.fori_loop(0, n_g, gbody, acc)

        acc = jnp.zeros((L,), jnp.float32)
        pend = fire(0, 0)
        for c in range(n_ch):
            p = c % 2
            for d_ in pend:
                d_.wait()
            if c + 1 < n_ch:
                pend = fire(c + 1, 1 - p)
            acc = compute(c, p, acc)

        part_v[...] = acc
        pltpu.sync_copy(part_v, out_hbm.at[pl.ds(wid * L, L)])

    return sc_fused


@functools.lru_cache(maxsize=None)
def _make_tc_mean(n, B):
    inv_b = 1.0 / B

    def body(x_ref, out_ref):
        out_ref[0, 0] = jnp.sum(x_ref[...]) * inv_b

    return pl.pallas_call(
        body,
        in_specs=[pl.BlockSpec(memory_space=pltpu.VMEM)],
        out_specs=pl.BlockSpec(memory_space=pltpu.SMEM),
        out_shape=jax.ShapeDtypeStruct((1, 1), jnp.float32),
    )


def kernel(positive_triples, negative_triples, entity_embeddings,
           relation_embeddings):
    B = positive_triples.shape[1]
    E, D = entity_embeddings.shape
    pack = _make_tc_pack(E, D, 1024)
    ent_p = pack(entity_embeddings.T)
    rel_p = pack(relation_embeddings.T)
    parts = _make_sc_fused(B, D)(
        positive_triples[0], positive_triples[1], positive_triples[2],
        negative_triples[0], negative_triples[1], negative_triples[2],
        ent_p, rel_p)
    tot = _make_tc_mean(parts.shape[0], B)(parts)
    return tot[0, 0]


# speedup vs baseline: 1.5180x; 1.5180x over previous
import functools

import jax
import jax.numpy as jnp
from jax import lax
from jax.experimental import pallas as pl
from jax.experimental.pallas import tpu as pltpu
from jax.experimental.pallas import tpu_sc as plsc

MARGIN = 1.0
EPS = 1e-8
CHUNK = 64


def _nrsqrt(x):
    xi = plsc.bitcast(x, jnp.int32)
    yi = jnp.int32(0x5F3759DF) - lax.shift_right_logical(xi, 1)
    y = plsc.bitcast(yi, jnp.float32)
    for _ in range(3):
        y = y * (1.5 - 0.5 * x * y * y)
    return y


@functools.lru_cache(maxsize=None)
def _make_tc_pack(E, D, bw):
    grid = (E + bw - 1) // bw
    h = bw // 2

    def body(x_ref, o_ref):
        x = x_ref[...].reshape(D, bw)
        y = jnp.swapaxes(x, 0, 1)
        o_ref[:, 0:D] = y[0:h]
        o_ref[:, D:2 * D] = y[h:bw]

    return pl.pallas_call(
        body,
        grid=(grid,),
        in_specs=[pl.BlockSpec((D // 8, 8, bw), lambda i: (0, 0, i))],
        out_specs=pl.BlockSpec((h, 2 * D), lambda i: (i, 0)),
        out_shape=jax.ShapeDtypeStruct((grid * h, 2 * D), jnp.float32),
    )


@functools.lru_cache(maxsize=None)
def _make_sc_fused(B, D, bw):
    hb = bw // 2
    hbit = hb.bit_length() - 1
    info = plsc.get_sparse_core_info()
    NC, NS, L = info.num_cores, info.num_subcores, info.num_lanes
    NW = NC * NS
    b_per_w = B // NW
    assert B % NW == 0 and b_per_w % CHUNK == 0 and CHUNK % L == 0
    n_ch = b_per_w // CHUNK
    n_g = CHUNK // L
    W = 2 * D
    mesh = plsc.VectorSubcoreMesh(core_axis_name="c", subcore_axis_name="s")

    row_scratch = [pltpu.VMEM((CHUNK, W), jnp.float32) for _ in range(12)]
    idx_scratch = ([pltpu.VMEM((b_per_w + L,), jnp.int32) for _ in range(6)]
                   + [pltpu.VMEM((b_per_w,), jnp.int32) for _ in range(6)])

    @functools.partial(
        pl.kernel,
        mesh=mesh,
        compiler_params=pltpu.CompilerParams(
            use_tc_tiling_on_sc=True, needs_layout_passes=False),
        out_type=jax.ShapeDtypeStruct((NW * L,), jnp.float32),
        scratch_types=idx_scratch + row_scratch + [
            pltpu.VMEM((12 * CHUNK,), jnp.float32),
            pltpu.VMEM((L,), jnp.float32),
            pltpu.SemaphoreType.DMA,
            pltpu.SemaphoreType.DMA,
        ],
    )
    def sc_fused(ph_hbm, pr_hbm, pt_hbm, nh_hbm, nr_hbm, nt_hbm,
                 ent_hbm, rel_hbm, out_hbm, *refs):
        idx_bufs = refs[0:6]
        tid_bufs = refs[6:12]
        bufs = (refs[12:18], refs[18:24])
        sums = refs[24]
        part_v = refs[25]
        sems = (refs[26], refs[27])
        wid = lax.axis_index("s") * NC + lax.axis_index("c")
        base = wid * b_per_w
        tables = (ent_hbm, rel_hbm, ent_hbm, ent_hbm, rel_hbm, ent_hbm)
        id_hbm = (ph_hbm, pr_hbm, pt_hbm, nh_hbm, nr_hbm, nt_hbm)

        for e in range(6):
            pltpu.sync_copy(id_hbm[e].at[pl.ds(base, b_per_w)],
                            idx_bufs[e].at[pl.ds(0, b_per_w)])

        def sbody(j, _):
            for e in range(6):
                v = idx_bufs[e][pl.ds(j * L, L)]
                hi = lax.shift_right_logical(v & jnp.int32(-bw), 1)
                tid_bufs[e][pl.ds(j * L, L)] = hi | (v & (hb - 1))
            return 0

        lax.fori_loop(0, b_per_w // L, sbody, 0)

        def fire(c, p):
            return [
                pltpu.async_copy(
                    tables[e].at[tid_bufs[e].at[pl.ds(c * CHUNK, CHUNK)]],
                    bufs[p][e], sems[p])
                for e in range(6)
            ]

        last = lax.iota(jnp.int32, L) == (L - 1)

        def compute(c, p, acc):
            def tbody(i, _):
                def ld(e):
                    v0 = idx_bufs[e][pl.ds(c * CHUNK + i, L)][0]
                    o = (lax.shift_right_logical(v0, hbit - (D.bit_length() - 1))
                         & jnp.int32(D))
                    return [bufs[p][e][i, pl.ds(o + k * L, L)]
                            for k in range(D // L)]

                h, r, t, h2, r2, t2 = (ld(e) for e in range(6))
                he = [v + EPS for v in h]
                te = [v + EPS for v in t]
                h2e = [v + EPS for v in h2]
                t2e = [v + EPS for v in t2]

                def red(a, b):
                    s = a[0] * b[0]
                    for k in range(1, len(a)):
                        s = s + a[k] * b[k]
                    return s

                terms = (red(he, he), red(te, te), red(r, r),
                         red(he, r), red(he, te), red(r, te),
                         red(h2e, h2e), red(t2e, t2e), red(r2, r2),
                         red(h2e, r2), red(h2e, t2e), red(r2, t2e))
                iv = jnp.full((L,), 0, jnp.int32) + i
                for m, v in enumerate(terms):
                    plsc.store_scatter(sums, [iv + m * CHUNK],
                                       plsc.cumsum(v), mask=last)
                return 0

            lax.fori_loop(0, CHUNK, tbody, 0)

            def gbody(g, acc):
                o = g * L
                (sh, st, sr, chr_, cht, crt,
                 sh2, st2, sr2, chr2, cht2, crt2) = (
                     sums[pl.ds(o + m * CHUNK, L)] for m in range(12))

                def score(sa, sb, sc_, cab, cac, cbc):
                    al = _nrsqrt(jnp.maximum(sa, 1e-24))
                    be = _nrsqrt(jnp.maximum(sb, 1e-24))
                    sq = (al * al * sa + be * be * sb + sc_
                          + 2.0 * al * cab - 2.0 * al * be * cac
                          - 2.0 * be * cbc)
                    sq = jnp.maximum(sq, 0.0)
                    return sq * _nrsqrt(jnp.maximum(sq, 1e-24))

                pos = score(sh, st, sr, chr_, cht, crt)
                neg = score(sh2, st2, sr2, chr2, cht2, crt2)
                return acc + jnp.maximum(0.0, MARGIN + pos - neg)

            return lax.fori_loop(0, n_g, gbody, acc)

        acc = jnp.zeros((L,), jnp.float32)
        pend = fire(0, 0)
        for c in range(n_ch):
            p = c % 2
            for d_ in pend:
                d_.wait()
            if c + 1 < n_ch:
                pend = fire(c + 1, 1 - p)
            acc = compute(c, p, acc)

        part_v[...] = acc
        pltpu.sync_copy(part_v, out_hbm.at[pl.ds(wid * L, L)])

    return sc_fused


@functools.lru_cache(maxsize=None)
def _make_tc_mean(n, B):
    inv_b = 1.0 / B

    def body(x_ref, out_ref):
        out_ref[0, 0] = jnp.sum(x_ref[...]) * inv_b

    return pl.pallas_call(
        body,
        in_specs=[pl.BlockSpec(memory_space=pltpu.VMEM)],
        out_specs=pl.BlockSpec(memory_space=pltpu.SMEM),
        out_shape=jax.ShapeDtypeStruct((1, 1), jnp.float32),
    )


def kernel(positive_triples, negative_triples, entity_embeddings,
           relation_embeddings):
    B = positive_triples.shape[1]
    E, D = entity_embeddings.shape
    pack = _make_tc_pack(E, D, 2048)
    ent_p = pack(entity_embeddings.T.reshape(D // 8, 8, E))
    rel_p = pack(relation_embeddings.T.reshape(D // 8, 8, E))
    parts = _make_sc_fused(B, D, 2048)(
        positive_triples[0], positive_triples[1], positive_triples[2],
        negative_triples[0], negative_triples[1], negative_triples[2],
        ent_p, rel_p)
    tot = _make_tc_mean(parts.shape[0], B)(parts)
    return tot[0, 0]


# speedup vs baseline: 2.0430x; 1.3459x over previous
import functools

import jax
import jax.numpy as jnp
from jax import lax
from jax.experimental import pallas as pl
from jax.experimental.pallas import tpu as pltpu
from jax.experimental.pallas import tpu_sc as plsc

MARGIN = 1.0
EPS = 1e-8
CHUNK = 64
BW = 4096


def _nrsqrt(x):
    xi = plsc.bitcast(x, jnp.int32)
    yi = jnp.int32(0x5F3759DF) - lax.shift_right_logical(xi, 1)
    y = plsc.bitcast(yi, jnp.float32)
    for _ in range(3):
        y = y * (1.5 - 0.5 * x * y * y)
    return y


@functools.lru_cache(maxsize=None)
def _make_tc_pack(E, D, bw):
    grid = (E + bw - 1) // bw
    h = bw // 2

    def body(x_ref, o_ref):
        x = x_ref[...].reshape(D, bw)
        y = jnp.swapaxes(x, 0, 1)
        o_ref[:, 0:D] = y[0:h]
        o_ref[:, D:2 * D] = y[h:bw]

    return pl.pallas_call(
        body,
        grid=(grid,),
        in_specs=[pl.BlockSpec((D // 8, 8, bw), lambda i: (0, 0, i))],
        out_specs=pl.BlockSpec((h, 2 * D), lambda i: (i, 0)),
        out_shape=jax.ShapeDtypeStruct((grid * h, 2 * D), jnp.float32),
    )


@functools.lru_cache(maxsize=None)
def _make_sc_fused(B, D, bw):
    hb = bw // 2
    hbit = hb.bit_length() - 1
    info = plsc.get_sparse_core_info()
    NC, NS, L = info.num_cores, info.num_subcores, info.num_lanes
    NW = NC * NS
    b_per_w = B // NW
    assert B % NW == 0 and b_per_w % CHUNK == 0 and CHUNK % L == 0
    n_ch = b_per_w // CHUNK
    n_g = CHUNK // L
    W = 2 * D
    mesh = plsc.VectorSubcoreMesh(core_axis_name="c", subcore_axis_name="s")

    row_scratch = [pltpu.VMEM((CHUNK, W), jnp.float32) for _ in range(12)]
    idx_scratch = ([pltpu.VMEM((b_per_w + L,), jnp.int32) for _ in range(6)]
                   + [pltpu.VMEM((b_per_w,), jnp.int32) for _ in range(6)])

    @functools.partial(
        pl.kernel,
        mesh=mesh,
        compiler_params=pltpu.CompilerParams(
            use_tc_tiling_on_sc=True, needs_layout_passes=False),
        out_type=jax.ShapeDtypeStruct((NW * L,), jnp.float32),
        scratch_types=idx_scratch + row_scratch + [
            pltpu.VMEM((12 * CHUNK,), jnp.float32),
            pltpu.VMEM((L,), jnp.float32),
            pltpu.SemaphoreType.DMA,
            pltpu.SemaphoreType.DMA,
        ],
    )
    def sc_fused(ph_hbm, pr_hbm, pt_hbm, nh_hbm, nr_hbm, nt_hbm,
                 ent_hbm, rel_hbm, out_hbm, *refs):
        idx_bufs = refs[0:6]
        tid_bufs = refs[6:12]
        bufs = (refs[12:18], refs[18:24])
        sums = refs[24]
        part_v = refs[25]
        sems = (refs[26], refs[27])
        wid = lax.axis_index("s") * NC + lax.axis_index("c")
        base = wid * b_per_w
        tables = (ent_hbm, rel_hbm, ent_hbm, ent_hbm, rel_hbm, ent_hbm)
        id_hbm = (ph_hbm, pr_hbm, pt_hbm, nh_hbm, nr_hbm, nt_hbm)

        for e in range(6):
            pltpu.sync_copy(id_hbm[e].at[pl.ds(base, b_per_w)],
                            idx_bufs[e].at[pl.ds(0, b_per_w)])

        def sbody(j, _):
            for e in range(6):
                v = idx_bufs[e][pl.ds(j * L, L)]
                hi = lax.shift_right_logical(v & jnp.int32(-bw), 1)
                tid_bufs[e][pl.ds(j * L, L)] = hi | (v & (hb - 1))
            return 0

        lax.fori_loop(0, b_per_w // L, sbody, 0)

        def fire(c, p):
            return [
                pltpu.async_copy(
                    tables[e].at[tid_bufs[e].at[pl.ds(c * CHUNK, CHUNK)]],
                    bufs[p][e], sems[p])
                for e in range(6)
            ]

        last = lax.iota(jnp.int32, L) == (L - 1)

        def compute(c, p, acc):
            def tbody(i, _):
                def ld(e):
                    v0 = idx_bufs[e][pl.ds(c * CHUNK + i, L)][0]
                    o = (lax.shift_right_logical(
                        v0, hbit - (D.bit_length() - 1)) & jnp.int32(D))
                    return [bufs[p][e][i, pl.ds(o + k * L, L)]
                            for k in range(D // L)]

                h, r, t, h2, r2, t2 = (ld(e) for e in range(6))
                he = [v + EPS for v in h]
                te = [v + EPS for v in t]
                h2e = [v + EPS for v in h2]
                t2e = [v + EPS for v in t2]

                def red(a, b):
                    s = a[0] * b[0]
                    for k in range(1, len(a)):
                        s = s + a[k] * b[k]
                    return s

                terms = (red(he, he), red(te, te), red(r, r),
                         red(he, r), red(he, te), red(r, te),
                         red(h2e, h2e), red(t2e, t2e), red(r2, r2),
                         red(h2e, r2), red(h2e, t2e), red(r2, t2e))
                iv = jnp.full((L,), 0, jnp.int32) + i
                for m, v in enumerate(terms):
                    plsc.store_scatter(sums, [iv + m * CHUNK],
                                       plsc.cumsum(v), mask=last)
                return 0

            lax.fori_loop(0, CHUNK, tbody, 0)

            def gbody(g, acc):
                o = g * L
                (sh, st, sr, chr_, cht, crt,
                 sh2, st2, sr2, chr2, cht2, crt2) = (
                     sums[pl.ds(o + m * CHUNK, L)] for m in range(12))

                def score(sa, sb, sc_, cab, cac, cbc):
                    al = _nrsqrt(jnp.maximum(sa, 1e-24))
                    be = _nrsqrt(jnp.maximum(sb, 1e-24))
                    sq = (al * al * sa + be * be * sb + sc_
                          + 2.0 * al * cab - 2.0 * al * be * cac
                          - 2.0 * be * cbc)
                    sq = jnp.maximum(sq, 0.0)
                    return sq * _nrsqrt(jnp.maximum(sq, 1e-24))

                pos = score(sh, st, sr, chr_, cht, crt)
                neg = score(sh2, st2, sr2, chr2, cht2, crt2)
                return acc + jnp.maximum(0.0, MARGIN + pos - neg)

            return lax.fori_loop(0, n_g, gbody, acc)

        acc = jnp.zeros((L,), jnp.float32)
        pend = fire(0, 0)
        for c in range(n_ch):
            p = c % 2
            for d_ in pend:
                d_.wait()
            if c + 1 < n_ch:
                pend = fire(c + 1, 1 - p)
            acc = compute(c, p, acc)

        part_v[...] = acc
        pltpu.sync_copy(part_v, out_hbm.at[pl.ds(wid * L, L)])

    return sc_fused


@functools.lru_cache(maxsize=None)
def _make_tc_mean(n, B):
    inv_b = 1.0 / B

    def body(x_ref, out_ref):
        out_ref[0, 0] = jnp.sum(x_ref[...]) * inv_b

    return pl.pallas_call(
        body,
        in_specs=[pl.BlockSpec(memory_space=pltpu.VMEM)],
        out_specs=pl.BlockSpec(memory_space=pltpu.SMEM),
        out_shape=jax.ShapeDtypeStruct((1, 1), jnp.float32),
    )


def kernel(positive_triples, negative_triples, entity_embeddings,
           relation_embeddings):
    B = positive_triples.shape[1]
    E, D = entity_embeddings.shape
    pack = _make_tc_pack(E, D, BW)
    ent_p = pack(entity_embeddings.T.reshape(D // 8, 8, E))
    rel_p = pack(relation_embeddings.T.reshape(D // 8, 8, E))
    parts = _make_sc_fused(B, D, BW)(
        positive_triples[0], positive_triples[1], positive_triples[2],
        negative_triples[0], negative_triples[1], negative_triples[2],
        ent_p, rel_p)
    tot = _make_tc_mean(parts.shape[0], B)(parts)
    return tot[0, 0]


# speedup vs baseline: 2.5213x; 1.2341x over previous
import functools

import jax
import jax.numpy as jnp
from jax import lax
from jax.experimental import pallas as pl
from jax.experimental.pallas import tpu as pltpu
from jax.experimental.pallas import tpu_sc as plsc

MARGIN = 1.0
EPS = 1e-8
CHUNK = 64
BW = 8192


def _nrsqrt(x):
    xi = plsc.bitcast(x, jnp.int32)
    yi = jnp.int32(0x5F3759DF) - lax.shift_right_logical(xi, 1)
    y = plsc.bitcast(yi, jnp.float32)
    for _ in range(3):
        y = y * (1.5 - 0.5 * x * y * y)
    return y


@functools.lru_cache(maxsize=None)
def _make_tc_pack(E, D, bw):
    grid = (E + bw - 1) // bw
    h = bw // 2

    def body(x_ref, o_ref):
        x = x_ref[...].reshape(D, bw)
        y = jnp.swapaxes(x, 0, 1)
        o_ref[:, 0:D] = y[0:h]
        o_ref[:, D:2 * D] = y[h:bw]

    return pl.pallas_call(
        body,
        grid=(grid,),
        in_specs=[pl.BlockSpec((D // 8, 8, bw), lambda i: (0, 0, i))],
        out_specs=pl.BlockSpec((h, 2 * D), lambda i: (i, 0)),
        out_shape=jax.ShapeDtypeStruct((grid * h, 2 * D), jnp.float32),
    )


@functools.lru_cache(maxsize=None)
def _make_sc_fused(B, D, bw):
    hb = bw // 2
    hbit = hb.bit_length() - 1
    info = plsc.get_sparse_core_info()
    NC, NS, L = info.num_cores, info.num_subcores, info.num_lanes
    NW = NC * NS
    b_per_w = B // NW
    assert B % NW == 0 and b_per_w % CHUNK == 0 and CHUNK % L == 0
    n_ch = b_per_w // CHUNK
    n_g = CHUNK // L
    W = 2 * D
    mesh = plsc.VectorSubcoreMesh(core_axis_name="c", subcore_axis_name="s")

    row_scratch = [pltpu.VMEM((CHUNK, W), jnp.float32) for _ in range(12)]
    idx_scratch = ([pltpu.VMEM((b_per_w + L,), jnp.int32) for _ in range(6)]
                   + [pltpu.VMEM((b_per_w,), jnp.int32) for _ in range(6)])

    @functools.partial(
        pl.kernel,
        mesh=mesh,
        compiler_params=pltpu.CompilerParams(
            use_tc_tiling_on_sc=True, needs_layout_passes=False),
        out_type=jax.ShapeDtypeStruct((NW * L,), jnp.float32),
        scratch_types=idx_scratch + row_scratch + [
            pltpu.VMEM((12 * CHUNK,), jnp.float32),
            pltpu.VMEM((L,), jnp.float32),
            pltpu.SemaphoreType.DMA,
            pltpu.SemaphoreType.DMA,
        ],
    )
    def sc_fused(ph_hbm, pr_hbm, pt_hbm, nh_hbm, nr_hbm, nt_hbm,
                 ent_hbm, rel_hbm, out_hbm, *refs):
        idx_bufs = refs[0:6]
        tid_bufs = refs[6:12]
        bufs = (refs[12:18], refs[18:24])
        sums = refs[24]
        part_v = refs[25]
        sems = (refs[26], refs[27])
        wid = lax.axis_index("s") * NC + lax.axis_index("c")
        base = wid * b_per_w
        tables = (ent_hbm, rel_hbm, ent_hbm, ent_hbm, rel_hbm, ent_hbm)
        id_hbm = (ph_hbm, pr_hbm, pt_hbm, nh_hbm, nr_hbm, nt_hbm)

        for e in range(6):
            pltpu.sync_copy(id_hbm[e].at[pl.ds(base, b_per_w)],
                            idx_bufs[e].at[pl.ds(0, b_per_w)])

        def sbody(j, _):
            for e in range(6):
                v = idx_bufs[e][pl.ds(j * L, L)]
                hi = lax.shift_right_logical(v & jnp.int32(-bw), 1)
                tid_bufs[e][pl.ds(j * L, L)] = hi | (v & (hb - 1))
            return 0

        lax.fori_loop(0, b_per_w // L, sbody, 0)

        def fire(c, p):
            return [
                pltpu.async_copy(
                    tables[e].at[tid_bufs[e].at[pl.ds(c * CHUNK, CHUNK)]],
                    bufs[p][e], sems[p])
                for e in range(6)
            ]

        last = lax.iota(jnp.int32, L) == (L - 1)

        def compute(c, p, acc):
            def tbody(i, _):
                def ld(e):
                    v0 = idx_bufs[e][pl.ds(c * CHUNK + i, L)][0]
                    o = (lax.shift_right_logical(
                        v0, hbit - (D.bit_length() - 1)) & jnp.int32(D))
                    return [bufs[p][e][i, pl.ds(o + k * L, L)]
                            for k in range(D // L)]

                h, r, t, h2, r2, t2 = (ld(e) for e in range(6))
                he = [v + EPS for v in h]
                te = [v + EPS for v in t]
                h2e = [v + EPS for v in h2]
                t2e = [v + EPS for v in t2]

                def red(a, b):
                    s = a[0] * b[0]
                    for k in range(1, len(a)):
                        s = s + a[k] * b[k]
                    return s

                terms = (red(he, he), red(te, te), red(r, r),
                         red(he, r), red(he, te), red(r, te),
                         red(h2e, h2e), red(t2e, t2e), red(r2, r2),
                         red(h2e, r2), red(h2e, t2e), red(r2, t2e))
                iv = jnp.full((L,), 0, jnp.int32) + i
                for m, v in enumerate(terms):
                    plsc.store_scatter(sums, [iv + m * CHUNK],
                                       plsc.cumsum(v), mask=last)
                return 0

            lax.fori_loop(0, CHUNK, tbody, 0)

            def gbody(g, acc):
                o = g * L
                (sh, st, sr, chr_, cht, crt,
                 sh2, st2, sr2, chr2, cht2, crt2) = (
                     sums[pl.ds(o + m * CHUNK, L)] for m in range(12))

                def score(sa, sb, sc_, cab, cac, cbc):
                    al = _nrsqrt(jnp.maximum(sa, 1e-24))
                    be = _nrsqrt(jnp.maximum(sb, 1e-24))
                    sq = (al * al * sa + be * be * sb + sc_
                          + 2.0 * al * cab - 2.0 * al * be * cac
                          - 2.0 * be * cbc)
                    sq = jnp.maximum(sq, 0.0)
                    return sq * _nrsqrt(jnp.maximum(sq, 1e-24))

                pos = score(sh, st, sr, chr_, cht, crt)
                neg = score(sh2, st2, sr2, chr2, cht2, crt2)
                return acc + jnp.maximum(0.0, MARGIN + pos - neg)

            return lax.fori_loop(0, n_g, gbody, acc)

        acc = jnp.zeros((L,), jnp.float32)
        pend = fire(0, 0)
        for c in range(n_ch):
            p = c % 2
            for d_ in pend:
                d_.wait()
            if c + 1 < n_ch:
                pend = fire(c + 1, 1 - p)
            acc = compute(c, p, acc)

        part_v[...] = acc
        pltpu.sync_copy(part_v, out_hbm.at[pl.ds(wid * L, L)])

    return sc_fused


@functools.lru_cache(maxsize=None)
def _make_tc_mean(n, B):
    inv_b = 1.0 / B

    def body(x_ref, out_ref):
        out_ref[0, 0] = jnp.sum(x_ref[...]) * inv_b

    return pl.pallas_call(
        body,
        in_specs=[pl.BlockSpec(memory_space=pltpu.VMEM)],
        out_specs=pl.BlockSpec(memory_space=pltpu.SMEM),
        out_shape=jax.ShapeDtypeStruct((1, 1), jnp.float32),
    )


def kernel(positive_triples, negative_triples, entity_embeddings,
           relation_embeddings):
    B = positive_triples.shape[1]
    E, D = entity_embeddings.shape
    pack = _make_tc_pack(E, D, BW)
    ent_p = pack(entity_embeddings.T.reshape(D // 8, 8, E))
    rel_p = pack(relation_embeddings.T.reshape(D // 8, 8, E))
    parts = _make_sc_fused(B, D, BW)(
        positive_triples[0], positive_triples[1], positive_triples[2],
        negative_triples[0], negative_triples[1], negative_triples[2],
        ent_p, rel_p)
    tot = _make_tc_mean(parts.shape[0], B)(parts)
    return tot[0, 0]


# speedup vs baseline: 2.8429x; 1.1276x over previous
import functools

import jax
import jax.numpy as jnp
from jax import lax
from jax.experimental import pallas as pl
from jax.experimental.pallas import tpu as pltpu
from jax.experimental.pallas import tpu_sc as plsc

MARGIN = 1.0
EPS = 1e-8
CHUNK = 64
BW = 16384


def _nrsqrt(x):
    xi = plsc.bitcast(x, jnp.int32)
    yi = jnp.int32(0x5F3759DF) - lax.shift_right_logical(xi, 1)
    y = plsc.bitcast(yi, jnp.float32)
    for _ in range(3):
        y = y * (1.5 - 0.5 * x * y * y)
    return y


@functools.lru_cache(maxsize=None)
def _make_tc_pack(E, D, bw):
    grid = (E + bw - 1) // bw
    h = bw // 2

    def body(x_ref, o_ref):
        x = x_ref[...].reshape(D, bw)
        y = jnp.swapaxes(x, 0, 1)
        o_ref[:, 0:D] = y[0:h]
        o_ref[:, D:2 * D] = y[h:bw]

    return pl.pallas_call(
        body,
        grid=(grid,),
        in_specs=[pl.BlockSpec((D // 8, 8, bw), lambda i: (0, 0, i))],
        out_specs=pl.BlockSpec((h, 2 * D), lambda i: (i, 0)),
        out_shape=jax.ShapeDtypeStruct((grid * h, 2 * D), jnp.float32),
    )


@functools.lru_cache(maxsize=None)
def _make_sc_fused(B, D, bw):
    hb = bw // 2
    hbit = hb.bit_length() - 1
    info = plsc.get_sparse_core_info()
    NC, NS, L = info.num_cores, info.num_subcores, info.num_lanes
    NW = NC * NS
    b_per_w = B // NW
    assert B % NW == 0 and b_per_w % CHUNK == 0 and CHUNK % L == 0
    n_ch = b_per_w // CHUNK
    n_g = CHUNK // L
    W = 2 * D
    mesh = plsc.VectorSubcoreMesh(core_axis_name="c", subcore_axis_name="s")

    row_scratch = [pltpu.VMEM((CHUNK, W), jnp.float32) for _ in range(12)]
    idx_scratch = ([pltpu.VMEM((b_per_w + L,), jnp.int32) for _ in range(6)]
                   + [pltpu.VMEM((b_per_w,), jnp.int32) for _ in range(6)])

    @functools.partial(
        pl.kernel,
        mesh=mesh,
        compiler_params=pltpu.CompilerParams(
            use_tc_tiling_on_sc=True, needs_layout_passes=False),
        out_type=jax.ShapeDtypeStruct((NW * L,), jnp.float32),
        scratch_types=idx_scratch + row_scratch + [
            pltpu.VMEM((12 * CHUNK,), jnp.float32),
            pltpu.VMEM((L,), jnp.float32),
            pltpu.SemaphoreType.DMA,
            pltpu.SemaphoreType.DMA,
        ],
    )
    def sc_fused(ph_hbm, pr_hbm, pt_hbm, nh_hbm, nr_hbm, nt_hbm,
                 ent_hbm, rel_hbm, out_hbm, *refs):
        idx_bufs = refs[0:6]
        tid_bufs = refs[6:12]
        bufs = (refs[12:18], refs[18:24])
        sums = refs[24]
        part_v = refs[25]
        sems = (refs[26], refs[27])
        wid = lax.axis_index("s") * NC + lax.axis_index("c")
        base = wid * b_per_w
        tables = (ent_hbm, rel_hbm, ent_hbm, ent_hbm, rel_hbm, ent_hbm)
        id_hbm = (ph_hbm, pr_hbm, pt_hbm, nh_hbm, nr_hbm, nt_hbm)

        for e in range(6):
            pltpu.sync_copy(id_hbm[e].at[pl.ds(base, b_per_w)],
                            idx_bufs[e].at[pl.ds(0, b_per_w)])

        def sbody(j, _):
            for e in range(6):
                v = idx_bufs[e][pl.ds(j * L, L)]
                hi = lax.shift_right_logical(v & jnp.int32(-bw), 1)
                tid_bufs[e][pl.ds(j * L, L)] = hi | (v & (hb - 1))
            return 0

        lax.fori_loop(0, b_per_w // L, sbody, 0)

        def fire(c, p):
            return [
                pltpu.async_copy(
                    tables[e].at[tid_bufs[e].at[pl.ds(c * CHUNK, CHUNK)]],
                    bufs[p][e], sems[p])
                for e in range(6)
            ]

        last = lax.iota(jnp.int32, L) == (L - 1)

        def compute(c, p, acc):
            def tbody(i, _):
                def ld(e):
                    v0 = idx_bufs[e][pl.ds(c * CHUNK + i, L)][0]
                    o = (lax.shift_right_logical(
                        v0, hbit - (D.bit_length() - 1)) & jnp.int32(D))
                    return [bufs[p][e][i, pl.ds(o + k * L, L)]
                            for k in range(D // L)]

                h, r, t, h2, r2, t2 = (ld(e) for e in range(6))
                he = [v + EPS for v in h]
                te = [v + EPS for v in t]
                h2e = [v + EPS for v in h2]
                t2e = [v + EPS for v in t2]

                def red(a, b):
                    s = a[0] * b[0]
                    for k in range(1, len(a)):
                        s = s + a[k] * b[k]
                    return s

                terms = (red(he, he), red(te, te), red(r, r),
                         red(he, r), red(he, te), red(r, te),
                         red(h2e, h2e), red(t2e, t2e), red(r2, r2),
                         red(h2e, r2), red(h2e, t2e), red(r2, t2e))
                iv = jnp.full((L,), 0, jnp.int32) + i
                for m, v in enumerate(terms):
                    plsc.store_scatter(sums, [iv + m * CHUNK],
                                       plsc.cumsum(v), mask=last)
                return 0

            lax.fori_loop(0, CHUNK, tbody, 0)

            def gbody(g, acc):
                o = g * L
                (sh, st, sr, chr_, cht, crt,
                 sh2, st2, sr2, chr2, cht2, crt2) = (
                     sums[pl.ds(o + m * CHUNK, L)] for m in range(12))

                def score(sa, sb, sc_, cab, cac, cbc):
                    al = _nrsqrt(jnp.maximum(sa, 1e-24))
                    be = _nrsqrt(jnp.maximum(sb, 1e-24))
                    sq = (al * al * sa + be * be * sb + sc_
                          + 2.0 * al * cab - 2.0 * al * be * cac
                          - 2.0 * be * cbc)
                    sq = jnp.maximum(sq, 0.0)
                    return sq * _nrsqrt(jnp.maximum(sq, 1e-24))

                pos = score(sh, st, sr, chr_, cht, crt)
                neg = score(sh2, st2, sr2, chr2, cht2, crt2)
                return acc + jnp.maximum(0.0, MARGIN + pos - neg)

            return lax.fori_loop(0, n_g, gbody, acc)

        acc = jnp.zeros((L,), jnp.float32)
        pend = fire(0, 0)
        for c in range(n_ch):
            p = c % 2
            for d_ in pend:
                d_.wait()
            if c + 1 < n_ch:
                pend = fire(c + 1, 1 - p)
            acc = compute(c, p, acc)

        part_v[...] = acc
        pltpu.sync_copy(part_v, out_hbm.at[pl.ds(wid * L, L)])

    return sc_fused


@functools.lru_cache(maxsize=None)
def _make_tc_mean(n, B):
    inv_b = 1.0 / B

    def body(x_ref, out_ref):
        out_ref[0, 0] = jnp.sum(x_ref[...]) * inv_b

    return pl.pallas_call(
        body,
        in_specs=[pl.BlockSpec(memory_space=pltpu.VMEM)],
        out_specs=pl.BlockSpec(memory_space=pltpu.SMEM),
        out_shape=jax.ShapeDtypeStruct((1, 1), jnp.float32),
    )


def kernel(positive_triples, negative_triples, entity_embeddings,
           relation_embeddings):
    B = positive_triples.shape[1]
    E, D = entity_embeddings.shape
    pack = _make_tc_pack(E, D, BW)
    ent_p = pack(entity_embeddings.T.reshape(D // 8, 8, E))
    rel_p = pack(relation_embeddings.T.reshape(D // 8, 8, E))
    parts = _make_sc_fused(B, D, BW)(
        positive_triples[0], positive_triples[1], positive_triples[2],
        negative_triples[0], negative_triples[1], negative_triples[2],
        ent_p, rel_p)
    tot = _make_tc_mean(parts.shape[0], B)(parts)
    return tot[0, 0]


# speedup vs baseline: 2.8472x; 1.0015x over previous
import functools

import jax
import jax.numpy as jnp
from jax import lax
from jax.experimental import pallas as pl
from jax.experimental.pallas import tpu as pltpu
from jax.experimental.pallas import tpu_sc as plsc

MARGIN = 1.0
EPS = 1e-8
CHUNK = 64
BW = 8192
SUB = 2


def _nrsqrt(x):
    xi = plsc.bitcast(x, jnp.int32)
    yi = jnp.int32(0x5F3759DF) - lax.shift_right_logical(xi, 1)
    y = plsc.bitcast(yi, jnp.float32)
    for _ in range(3):
        y = y * (1.5 - 0.5 * x * y * y)
    return y


@functools.lru_cache(maxsize=None)
def _make_tc_pack(E, D, bw, S):
    grid = (E + S * bw - 1) // (S * bw)
    h = bw // 2

    def body(x_ref, o_ref):
        for s in range(S):
            x = x_ref[:, :, s * bw:(s + 1) * bw].reshape(D, bw)
            y = jnp.swapaxes(x, 0, 1)
            o_ref[s * h:(s + 1) * h, 0:D] = y[0:h]
            o_ref[s * h:(s + 1) * h, D:2 * D] = y[h:bw]

    return pl.pallas_call(
        body,
        grid=(grid,),
        in_specs=[pl.BlockSpec((D // 8, 8, S * bw), lambda i: (0, 0, i))],
        out_specs=pl.BlockSpec((S * h, 2 * D), lambda i: (i, 0)),
        out_shape=jax.ShapeDtypeStruct((grid * S * h, 2 * D), jnp.float32),
    )


@functools.lru_cache(maxsize=None)
def _make_sc_fused(B, D, bw):
    hb = bw // 2
    hbit = hb.bit_length() - 1
    info = plsc.get_sparse_core_info()
    NC, NS, L = info.num_cores, info.num_subcores, info.num_lanes
    NW = NC * NS
    b_per_w = B // NW
    assert B % NW == 0 and b_per_w % CHUNK == 0 and CHUNK % L == 0
    n_ch = b_per_w // CHUNK
    n_g = CHUNK // L
    W = 2 * D
    mesh = plsc.VectorSubcoreMesh(core_axis_name="c", subcore_axis_name="s")

    row_scratch = [pltpu.VMEM((CHUNK, W), jnp.float32) for _ in range(12)]
    idx_scratch = ([pltpu.VMEM((b_per_w + L,), jnp.int32) for _ in range(6)]
                   + [pltpu.VMEM((b_per_w,), jnp.int32) for _ in range(6)])

    @functools.partial(
        pl.kernel,
        mesh=mesh,
        compiler_params=pltpu.CompilerParams(
            use_tc_tiling_on_sc=True, needs_layout_passes=False),
        out_type=jax.ShapeDtypeStruct((NW * L,), jnp.float32),
        scratch_types=idx_scratch + row_scratch + [
            pltpu.VMEM((12 * CHUNK,), jnp.float32),
            pltpu.VMEM((L,), jnp.float32),
            pltpu.SemaphoreType.DMA,
            pltpu.SemaphoreType.DMA,
        ],
    )
    def sc_fused(ph_hbm, pr_hbm, pt_hbm, nh_hbm, nr_hbm, nt_hbm,
                 ent_hbm, rel_hbm, out_hbm, *refs):
        idx_bufs = refs[0:6]
        tid_bufs = refs[6:12]
        bufs = (refs[12:18], refs[18:24])
        sums = refs[24]
        part_v = refs[25]
        sems = (refs[26], refs[27])
        wid = lax.axis_index("s") * NC + lax.axis_index("c")
        base = wid * b_per_w
        tables = (ent_hbm, rel_hbm, ent_hbm, ent_hbm, rel_hbm, ent_hbm)
        id_hbm = (ph_hbm, pr_hbm, pt_hbm, nh_hbm, nr_hbm, nt_hbm)

        for e in range(6):
            pltpu.sync_copy(id_hbm[e].at[pl.ds(base, b_per_w)],
                            idx_bufs[e].at[pl.ds(0, b_per_w)])

        def sbody(j, _):
            for e in range(6):
                v = idx_bufs[e][pl.ds(j * L, L)]
                hi = lax.shift_right_logical(v & jnp.int32(-bw), 1)
                tid_bufs[e][pl.ds(j * L, L)] = hi | (v & (hb - 1))
            return 0

        lax.fori_loop(0, b_per_w // L, sbody, 0)

        def fire(c, p):
            return [
                pltpu.async_copy(
                    tables[e].at[tid_bufs[e].at[pl.ds(c * CHUNK, CHUNK)]],
                    bufs[p][e], sems[p])
                for e in range(6)
            ]

        last = lax.iota(jnp.int32, L) == (L - 1)

        def compute(c, p, acc):
            def tbody(i, _):
                def ld(e):
                    v0 = idx_bufs[e][pl.ds(c * CHUNK + i, L)][0]
                    o = (lax.shift_right_logical(
                        v0, hbit - (D.bit_length() - 1)) & jnp.int32(D))
                    return [bufs[p][e][i, pl.ds(o + k * L, L)]
                            for k in range(D // L)]

                h, r, t, h2, r2, t2 = (ld(e) for e in range(6))
                he = [v + EPS for v in h]
                te = [v + EPS for v in t]
                h2e = [v + EPS for v in h2]
                t2e = [v + EPS for v in t2]

                def red(a, b):
                    s = a[0] * b[0]
                    for k in range(1, len(a)):
                        s = s + a[k] * b[k]
                    return s

                terms = (red(he, he), red(te, te), red(r, r),
                         red(he, r), red(he, te), red(r, te),
                         red(h2e, h2e), red(t2e, t2e), red(r2, r2),
                         red(h2e, r2), red(h2e, t2e), red(r2, t2e))
                iv = jnp.full((L,), 0, jnp.int32) + i
                for m, v in enumerate(terms):
                    plsc.store_scatter(sums, [iv + m * CHUNK],
                                       plsc.cumsum(v), mask=last)
                return 0

            lax.fori_loop(0, CHUNK, tbody, 0)

            def gbody(g, acc):
                o = g * L
                (sh, st, sr, chr_, cht, crt,
                 sh2, st2, sr2, chr2, cht2, crt2) = (
                     sums[pl.ds(o + m * CHUNK, L)] for m in range(12))

                def score(sa, sb, sc_, cab, cac, cbc):
                    al = _nrsqrt(jnp.maximum(sa, 1e-24))
                    be = _nrsqrt(jnp.maximum(sb, 1e-24))
                    sq = (al * al * sa + be * be * sb + sc_
                          + 2.0 * al * cab - 2.0 * al * be * cac
                          - 2.0 * be * cbc)
                    sq = jnp.maximum(sq, 0.0)
                    return sq * _nrsqrt(jnp.maximum(sq, 1e-24))

                pos = score(sh, st, sr, chr_, cht, crt)
                neg = score(sh2, st2, sr2, chr2, cht2, crt2)
                return acc + jnp.maximum(0.0, MARGIN + pos - neg)

            return lax.fori_loop(0, n_g, gbody, acc)

        acc = jnp.zeros((L,), jnp.float32)
        pend = fire(0, 0)
        for c in range(n_ch):
            p = c % 2
            for d_ in pend:
                d_.wait()
            if c + 1 < n_ch:
                pend = fire(c + 1, 1 - p)
            acc = compute(c, p, acc)

        part_v[...] = acc
        pltpu.sync_copy(part_v, out_hbm.at[pl.ds(wid * L, L)])

    return sc_fused


@functools.lru_cache(maxsize=None)
def _make_tc_mean(n, B):
    inv_b = 1.0 / B

    def body(x_ref, out_ref):
        out_ref[0, 0] = jnp.sum(x_ref[...]) * inv_b

    return pl.pallas_call(
        body,
        in_specs=[pl.BlockSpec(memory_space=pltpu.VMEM)],
        out_specs=pl.BlockSpec(memory_space=pltpu.SMEM),
        out_shape=jax.ShapeDtypeStruct((1, 1), jnp.float32),
    )


def kernel(positive_triples, negative_triples, entity_embeddings,
           relation_embeddings):
    B = positive_triples.shape[1]
    E, D = entity_embeddings.shape
    pack = _make_tc_pack(E, D, BW, SUB)
    ent_p = pack(entity_embeddings.T.reshape(D // 8, 8, E))
    rel_p = pack(relation_embeddings.T.reshape(D // 8, 8, E))
    parts = _make_sc_fused(B, D, BW)(
        positive_triples[0], positive_triples[1], positive_triples[2],
        negative_triples[0], negative_triples[1], negative_triples[2],
        ent_p, rel_p)
    tot = _make_tc_mean(parts.shape[0], B)(parts)
    return tot[0, 0]


# speedup vs baseline: 3.0201x; 1.0607x over previous
import functools

import jax
import jax.numpy as jnp
from jax import lax
from jax.experimental import pallas as pl
from jax.experimental.pallas import tpu as pltpu
from jax.experimental.pallas import tpu_sc as plsc

MARGIN = 1.0
EPS = 1e-8
CHUNK = 64
BW = 8192
SUB = 4


def _nrsqrt(x):
    xi = plsc.bitcast(x, jnp.int32)
    yi = jnp.int32(0x5F3759DF) - lax.shift_right_logical(xi, 1)
    y = plsc.bitcast(yi, jnp.float32)
    for _ in range(3):
        y = y * (1.5 - 0.5 * x * y * y)
    return y


@functools.lru_cache(maxsize=None)
def _make_tc_pack(E, D, bw, S):
    grid = (E + S * bw - 1) // (S * bw)
    h = bw // 2

    def body(x_ref, o_ref):
        for s in range(S):
            x = x_ref[:, :, s * bw:(s + 1) * bw].reshape(D, bw)
            y = jnp.swapaxes(x, 0, 1)
            o_ref[s * h:(s + 1) * h, 0:D] = y[0:h]
            o_ref[s * h:(s + 1) * h, D:2 * D] = y[h:bw]

    return pl.pallas_call(
        body,
        grid=(grid,),
        in_specs=[pl.BlockSpec((D // 8, 8, S * bw), lambda i: (0, 0, i))],
        out_specs=pl.BlockSpec((S * h, 2 * D), lambda i: (i, 0)),
        out_shape=jax.ShapeDtypeStruct((grid * S * h, 2 * D), jnp.float32),
    )


@functools.lru_cache(maxsize=None)
def _make_sc_fused(B, D, bw):
    hb = bw // 2
    hbit = hb.bit_length() - 1
    info = plsc.get_sparse_core_info()
    NC, NS, L = info.num_cores, info.num_subcores, info.num_lanes
    NW = NC * NS
    b_per_w = B // NW
    assert B % NW == 0 and b_per_w % CHUNK == 0 and CHUNK % L == 0
    n_ch = b_per_w // CHUNK
    n_g = CHUNK // L
    W = 2 * D
    mesh = plsc.VectorSubcoreMesh(core_axis_name="c", subcore_axis_name="s")

    row_scratch = [pltpu.VMEM((CHUNK, W), jnp.float32) for _ in range(12)]
    idx_scratch = ([pltpu.VMEM((b_per_w + L,), jnp.int32) for _ in range(6)]
                   + [pltpu.VMEM((b_per_w,), jnp.int32) for _ in range(6)])

    @functools.partial(
        pl.kernel,
        mesh=mesh,
        compiler_params=pltpu.CompilerParams(
            use_tc_tiling_on_sc=True, needs_layout_passes=False),
        out_type=jax.ShapeDtypeStruct((NW * L,), jnp.float32),
        scratch_types=idx_scratch + row_scratch + [
            pltpu.VMEM((12 * CHUNK,), jnp.float32),
            pltpu.VMEM((L,), jnp.float32),
            pltpu.SemaphoreType.DMA,
            pltpu.SemaphoreType.DMA,
        ],
    )
    def sc_fused(ph_hbm, pr_hbm, pt_hbm, nh_hbm, nr_hbm, nt_hbm,
                 ent_hbm, rel_hbm, out_hbm, *refs):
        idx_bufs = refs[0:6]
        tid_bufs = refs[6:12]
        bufs = (refs[12:18], refs[18:24])
        sums = refs[24]
        part_v = refs[25]
        sems = (refs[26], refs[27])
        wid = lax.axis_index("s") * NC + lax.axis_index("c")
        base = wid * b_per_w
        tables = (ent_hbm, rel_hbm, ent_hbm, ent_hbm, rel_hbm, ent_hbm)
        id_hbm = (ph_hbm, pr_hbm, pt_hbm, nh_hbm, nr_hbm, nt_hbm)

        for e in range(6):
            pltpu.sync_copy(id_hbm[e].at[pl.ds(base, b_per_w)],
                            idx_bufs[e].at[pl.ds(0, b_per_w)])

        def sbody(j, _):
            for e in range(6):
                v = idx_bufs[e][pl.ds(j * L, L)]
                hi = lax.shift_right_logical(v & jnp.int32(-bw), 1)
                tid_bufs[e][pl.ds(j * L, L)] = hi | (v & (hb - 1))
            return 0

        lax.fori_loop(0, b_per_w // L, sbody, 0)

        def fire(c, p):
            return [
                pltpu.async_copy(
                    tables[e].at[tid_bufs[e].at[pl.ds(c * CHUNK, CHUNK)]],
                    bufs[p][e], sems[p])
                for e in range(6)
            ]

        last = lax.iota(jnp.int32, L) == (L - 1)

        def compute(c, p, acc):
            def tbody(i, _):
                def ld(e):
                    v0 = idx_bufs[e][pl.ds(c * CHUNK + i, L)][0]
                    o = (lax.shift_right_logical(
                        v0, hbit - (D.bit_length() - 1)) & jnp.int32(D))
                    return [bufs[p][e][i, pl.ds(o + k * L, L)]
                            for k in range(D // L)]

                h, r, t, h2, r2, t2 = (ld(e) for e in range(6))
                he = [v + EPS for v in h]
                te = [v + EPS for v in t]
                h2e = [v + EPS for v in h2]
                t2e = [v + EPS for v in t2]

                def red(a, b):
                    s = a[0] * b[0]
                    for k in range(1, len(a)):
                        s = s + a[k] * b[k]
                    return s

                terms = (red(he, he), red(te, te), red(r, r),
                         red(he, r), red(he, te), red(r, te),
                         red(h2e, h2e), red(t2e, t2e), red(r2, r2),
                         red(h2e, r2), red(h2e, t2e), red(r2, t2e))
                iv = jnp.full((L,), 0, jnp.int32) + i
                for m, v in enumerate(terms):
                    plsc.store_scatter(sums, [iv + m * CHUNK],
                                       plsc.cumsum(v), mask=last)
                return 0

            lax.fori_loop(0, CHUNK, tbody, 0)

            def gbody(g, acc):
                o = g * L
                (sh, st, sr, chr_, cht, crt,
                 sh2, st2, sr2, chr2, cht2, crt2) = (
                     sums[pl.ds(o + m * CHUNK, L)] for m in range(12))

                def score(sa, sb, sc_, cab, cac, cbc):
                    al = _nrsqrt(jnp.maximum(sa, 1e-24))
                    be = _nrsqrt(jnp.maximum(sb, 1e-24))
                    sq = (al * al * sa + be * be * sb + sc_
                          + 2.0 * al * cab - 2.0 * al * be * cac
                          - 2.0 * be * cbc)
                    sq = jnp.maximum(sq, 0.0)
                    return sq * _nrsqrt(jnp.maximum(sq, 1e-24))

                pos = score(sh, st, sr, chr_, cht, crt)
                neg = score(sh2, st2, sr2, chr2, cht2, crt2)
                return acc + jnp.maximum(0.0, MARGIN + pos - neg)

            return lax.fori_loop(0, n_g, gbody, acc)

        acc = jnp.zeros((L,), jnp.float32)
        pend = fire(0, 0)
        for c in range(n_ch):
            p = c % 2
            for d_ in pend:
                d_.wait()
            if c + 1 < n_ch:
                pend = fire(c + 1, 1 - p)
            acc = compute(c, p, acc)

        part_v[...] = acc
        pltpu.sync_copy(part_v, out_hbm.at[pl.ds(wid * L, L)])

    return sc_fused


@functools.lru_cache(maxsize=None)
def _make_tc_mean(n, B):
    inv_b = 1.0 / B

    def body(x_ref, out_ref):
        out_ref[0, 0] = jnp.sum(x_ref[...]) * inv_b

    return pl.pallas_call(
        body,
        in_specs=[pl.BlockSpec(memory_space=pltpu.VMEM)],
        out_specs=pl.BlockSpec(memory_space=pltpu.SMEM),
        out_shape=jax.ShapeDtypeStruct((1, 1), jnp.float32),
    )


def kernel(positive_triples, negative_triples, entity_embeddings,
           relation_embeddings):
    B = positive_triples.shape[1]
    E, D = entity_embeddings.shape
    pack = _make_tc_pack(E, D, BW, SUB)
    ent_p = pack(entity_embeddings.T.reshape(D // 8, 8, E))
    rel_p = pack(relation_embeddings.T.reshape(D // 8, 8, E))
    parts = _make_sc_fused(B, D, BW)(
        positive_triples[0], positive_triples[1], positive_triples[2],
        negative_triples[0], negative_triples[1], negative_triples[2],
        ent_p, rel_p)
    tot = _make_tc_mean(parts.shape[0], B)(parts)
    return tot[0, 0]
